# baseline (speedup 1.0000x reference)
import functools

import jax
import jax.numpy as jnp
from jax import lax
from jax.experimental import pallas as pl
from jax.experimental.pallas import tpu as pltpu
from jax.experimental.pallas import tpu_sc as plsc

N_NODES = 10000
N_EDGES = 320000
D = 128

NC = 2
NS = 16
NW = NC * NS
CHUNK = 128
NCH = 80
EPW = NCH * CHUNK
N_ACC = N_NODES + 240
ROWS_PER_SUB = 624
TAIL_BASE = NS * ROWS_PER_SUB
TAIL = N_NODES - TAIL_BASE

_mesh = plsc.VectorSubcoreMesh(core_axis_name="c", subcore_axis_name="s")


@functools.partial(
    pl.kernel,
    mesh=_mesh,
    out_type=jax.ShapeDtypeStruct((2, N_NODES, D), jnp.float32),
    scratch_types=[
        pltpu.VMEM((4, CHUNK), jnp.int32),
        pltpu.VMEM((4, CHUNK), jnp.int32),
        pltpu.VMEM((CHUNK, D), jnp.float32),
        pltpu.VMEM((CHUNK, D), jnp.float32),
        pltpu.VMEM_SHARED((N_ACC, D), jnp.float32),
        pltpu.SemaphoreType.DMA,
        pltpu.SemaphoreType.DMA,
        pltpu.SemaphoreType.DMA,
        pltpu.SemaphoreType.DMA,
        pltpu.SemaphoreType.DMA,
        pltpu.SemaphoreType.DMA,
    ],
)
def _edge_agg(src_hbm, dst_hbm, h_hbm, out_hbm, sidxb, didxb, rows0, rows1,
              acc, semg0, semg1, semi0, semi1, semi2, semi3):
    cid = lax.axis_index("c")
    sid = lax.axis_index("s")
    wid = cid * NS + sid
    ebase = wid * EPW

    base = sid * ROWS_PER_SUB
    pltpu.sync_copy(h_hbm.at[pl.ds(base, ROWS_PER_SUB)],
                    acc.at[pl.ds(base, ROWS_PER_SUB)])

    @pl.when(sid == NS - 1)
    def _():
        pltpu.sync_copy(h_hbm.at[pl.ds(TAIL_BASE, TAIL)],
                        acc.at[pl.ds(TAIL_BASE, TAIL)])

    plsc.subcore_barrier()

    rowsb = (rows0, rows1)
    semg = (semg0, semg1)
    semi = (semi0, semi1, semi2, semi3)

    def _off(i):
        return pl.ds(pl.multiple_of(ebase + i * CHUNK, 8), CHUNK)

    def _idx_load(i, q):
        pltpu.async_copy(src_hbm.at[_off(i)], sidxb.at[q], semi[q])
        pltpu.async_copy(dst_hbm.at[_off(i)], didxb.at[q], semi[q])

    def _idx_wait(i, q):
        pltpu.make_async_copy(src_hbm.at[_off(i)], sidxb.at[q],
                              semi[q]).wait()
        pltpu.make_async_copy(dst_hbm.at[_off(i)], didxb.at[q],
                              semi[q]).wait()

    def _gather(q, r):
        pltpu.async_copy(h_hbm.at[sidxb.at[q]], rowsb[r], semg[r])

    def _gwait(r):
        pltpu.make_async_copy(h_hbm.at[sidxb.at[0]], rowsb[r],
                              semg[r]).wait()

    for q in range(4):
        _idx_load(q, q)
    _idx_wait(0, 0)
    _gather(0, 0)
    _idx_wait(1, 1)
    _gather(1, 1)

    def body(j, carry):
        c0 = 4 * j
        for k in range(4):
            i = c0 + k
            r = k % 2
            _gwait(r)
            pltpu.sync_copy(rowsb[r], acc.at[didxb.at[k]], add=True)

            @pl.when(i + 4 < NCH)
            def _():
                _idx_load(i + 4, k)

            @pl.when(i + 2 < NCH)
            def _():
                _idx_wait(i + 2, (k + 2) % 4)
                _gather((k + 2) % 4, r)

        return carry

    lax.fori_loop(0, NCH // 4, body, 0)

    plsc.subcore_barrier()
    pltpu.sync_copy(acc.at[pl.ds(base, ROWS_PER_SUB)],
                    out_hbm.at[cid, pl.ds(base, ROWS_PER_SUB)])

    @pl.when(sid == NS - 1)
    def _():
        pltpu.sync_copy(acc.at[pl.ds(TAIL_BASE, TAIL)],
                        out_hbm.at[cid, pl.ds(TAIL_BASE, TAIL)])


_BLK = 1000
_GRID = N_NODES // _BLK


def _mm(x, W, b):
    def body(x_ref, w_ref, b_ref, o_ref):
        o_ref[...] = jnp.dot(x_ref[...], w_ref[...],
                             preferred_element_type=jnp.float32) + b_ref[...]

    return pl.pallas_call(
        body,
        grid=(_GRID,),
        in_specs=[pl.BlockSpec((_BLK, D), lambda i: (i, 0)),
                  pl.BlockSpec((D, D), lambda i: (0, 0)),
                  pl.BlockSpec((1, D), lambda i: (0, 0))],
        out_specs=pl.BlockSpec((_BLK, D), lambda i: (i, 0)),
        out_shape=jax.ShapeDtypeStruct((N_NODES, D), jnp.float32),
    )(x, W, b.reshape(1, D))


def _combine_mm(p0, p1, h, W, b):
    def body(p0_ref, p1_ref, h_ref, w_ref, b_ref, o_ref):
        z = jnp.maximum(p0_ref[...] + p1_ref[...] - h_ref[...], 0.0)
        o_ref[...] = jnp.dot(z, w_ref[...],
                             preferred_element_type=jnp.float32) + b_ref[...]

    return pl.pallas_call(
        body,
        grid=(_GRID,),
        in_specs=[pl.BlockSpec((_BLK, D), lambda i: (i, 0)),
                  pl.BlockSpec((_BLK, D), lambda i: (i, 0)),
                  pl.BlockSpec((_BLK, D), lambda i: (i, 0)),
                  pl.BlockSpec((D, D), lambda i: (0, 0)),
                  pl.BlockSpec((1, D), lambda i: (0, 0))],
        out_specs=pl.BlockSpec((_BLK, D), lambda i: (i, 0)),
        out_shape=jax.ShapeDtypeStruct((N_NODES, D), jnp.float32),
    )(p0, p1, h, W, b.reshape(1, D))


def _combine_relu(p0, p1, h):
    def body(p0_ref, p1_ref, h_ref, o_ref):
        o_ref[...] = jnp.maximum(p0_ref[...] + p1_ref[...] - h_ref[...], 0.0)

    return pl.pallas_call(
        body,
        grid=(_GRID,),
        in_specs=[pl.BlockSpec((_BLK, D), lambda i: (i, 0)),
                  pl.BlockSpec((_BLK, D), lambda i: (i, 0)),
                  pl.BlockSpec((_BLK, D), lambda i: (i, 0))],
        out_specs=pl.BlockSpec((_BLK, D), lambda i: (i, 0)),
        out_shape=jax.ShapeDtypeStruct((N_NODES, D), jnp.float32),
    )(p0, p1, h)


def kernel(edge_index, node_feats, W1, b1, W2, b2):
    real_epw = N_EDGES // NW
    pad = EPW - real_epw
    src = jnp.pad(edge_index[0].astype(jnp.int32).reshape(NW, real_epw),
                  ((0, 0), (0, pad))).reshape(-1)
    dst_fill = jnp.broadcast_to(N_NODES + jnp.arange(pad, dtype=jnp.int32),
                                (NW, pad))
    dst = jnp.concatenate(
        [edge_index[1].astype(jnp.int32).reshape(NW, real_epw), dst_fill],
        axis=1).reshape(-1)
    h1 = _mm(node_feats, W1, b1)
    p = _edge_agg(src, dst, h1)
    h2 = _combine_mm(p[0], p[1], h1, W2, b2)
    q = _edge_agg(src, dst, h2)
    return _combine_relu(q[0], q[1], h2)


# speedup vs baseline: 2.7533x; 2.7533x over previous
import functools

import jax
import jax.numpy as jnp
from jax import lax
from jax.experimental import pallas as pl
from jax.experimental.pallas import tpu as pltpu
from jax.experimental.pallas import tpu_sc as plsc

N_NODES = 10000
N_EDGES = 320000
D = 128

NC = 2
NS = 16
NW = NC * NS
CHUNK = 80
NCH = 125
EPW = NCH * CHUNK
N_ACC = N_NODES
ROWS_PER_SUB = 624
TAIL_BASE = NS * ROWS_PER_SUB
TAIL = N_NODES - TAIL_BASE

_mesh = plsc.VectorSubcoreMesh(core_axis_name="c", subcore_axis_name="s")


@functools.partial(
    pl.kernel,
    mesh=_mesh,
    out_type=jax.ShapeDtypeStruct((2, N_NODES, D), jnp.float32),
    scratch_types=[
        pltpu.VMEM((EPW,), jnp.int32),
        pltpu.VMEM((EPW,), jnp.int32),
        pltpu.VMEM((CHUNK, D), jnp.float32),
        pltpu.VMEM((CHUNK, D), jnp.float32),
        pltpu.VMEM_SHARED((N_ACC, D), jnp.float32),
        pltpu.SemaphoreType.DMA,
        pltpu.SemaphoreType.DMA,
    ],
)
def _edge_agg(src_hbm, dst_hbm, h_hbm, out_hbm, sidx, didx, rows0, rows1,
              acc, sem0, sem1):
    cid = lax.axis_index("c")
    sid = lax.axis_index("s")
    wid = cid * NS + sid
    ebase = wid * EPW

    pltpu.sync_copy(src_hbm.at[pl.ds(ebase, EPW)], sidx)
    pltpu.sync_copy(dst_hbm.at[pl.ds(ebase, EPW)], didx)

    base = sid * ROWS_PER_SUB
    pltpu.sync_copy(h_hbm.at[pl.ds(base, ROWS_PER_SUB)],
                    acc.at[pl.ds(base, ROWS_PER_SUB)])

    @pl.when(sid == NS - 1)
    def _():
        pltpu.sync_copy(h_hbm.at[pl.ds(TAIL_BASE, TAIL)],
                        acc.at[pl.ds(TAIL_BASE, TAIL)])

    plsc.subcore_barrier()

    def _chunk(ref, i):
        return ref.at[pl.ds(pl.multiple_of(i * CHUNK, 8), CHUNK)]

    pltpu.async_copy(h_hbm.at[_chunk(sidx, 0)], rows0, sem0)

    def body(j, carry):
        i0 = 2 * j
        pltpu.async_copy(h_hbm.at[_chunk(sidx, i0 + 1)], rows1, sem1)
        pltpu.make_async_copy(h_hbm.at[_chunk(sidx, i0)], rows0,
                              sem0).wait()
        pltpu.sync_copy(rows0, acc.at[_chunk(didx, i0)], add=True)
        pltpu.async_copy(h_hbm.at[_chunk(sidx, i0 + 2)], rows0, sem0)
        pltpu.make_async_copy(h_hbm.at[_chunk(sidx, i0 + 1)], rows1,
                              sem1).wait()
        pltpu.sync_copy(rows1, acc.at[_chunk(didx, i0 + 1)], add=True)
        return carry

    lax.fori_loop(0, (NCH - 1) // 2, body, 0)
    pltpu.make_async_copy(h_hbm.at[_chunk(sidx, NCH - 1)], rows0,
                          sem0).wait()
    pltpu.sync_copy(rows0, acc.at[_chunk(didx, NCH - 1)], add=True)

    plsc.subcore_barrier()
    pltpu.sync_copy(acc.at[pl.ds(base, ROWS_PER_SUB)],
                    out_hbm.at[cid, pl.ds(base, ROWS_PER_SUB)])

    @pl.when(sid == NS - 1)
    def _():
        pltpu.sync_copy(acc.at[pl.ds(TAIL_BASE, TAIL)],
                        out_hbm.at[cid, pl.ds(TAIL_BASE, TAIL)])


_BLK = 1000
_GRID = N_NODES // _BLK


def _mm(x, W, b):
    def body(x_ref, w_ref, b_ref, o_ref):
        o_ref[...] = jnp.dot(x_ref[...], w_ref[...],
                             preferred_element_type=jnp.float32) + b_ref[...]

    return pl.pallas_call(
        body,
        grid=(_GRID,),
        in_specs=[pl.BlockSpec((_BLK, D), lambda i: (i, 0)),
                  pl.BlockSpec((D, D), lambda i: (0, 0)),
                  pl.BlockSpec((1, D), lambda i: (0, 0))],
        out_specs=pl.BlockSpec((_BLK, D), lambda i: (i, 0)),
        out_shape=jax.ShapeDtypeStruct((N_NODES, D), jnp.float32),
    )(x, W, b.reshape(1, D))


def _combine_mm(p0, p1, h, W, b):
    def body(p0_ref, p1_ref, h_ref, w_ref, b_ref, o_ref):
        z = jnp.maximum(p0_ref[...] + p1_ref[...] - h_ref[...], 0.0)
        o_ref[...] = jnp.dot(z, w_ref[...],
                             preferred_element_type=jnp.float32) + b_ref[...]

    return pl.pallas_call(
        body,
        grid=(_GRID,),
        in_specs=[pl.BlockSpec((_BLK, D), lambda i: (i, 0)),
                  pl.BlockSpec((_BLK, D), lambda i: (i, 0)),
                  pl.BlockSpec((_BLK, D), lambda i: (i, 0)),
                  pl.BlockSpec((D, D), lambda i: (0, 0)),
                  pl.BlockSpec((1, D), lambda i: (0, 0))],
        out_specs=pl.BlockSpec((_BLK, D), lambda i: (i, 0)),
        out_shape=jax.ShapeDtypeStruct((N_NODES, D), jnp.float32),
    )(p0, p1, h, W, b.reshape(1, D))


def _combine_relu(p0, p1, h):
    def body(p0_ref, p1_ref, h_ref, o_ref):
        o_ref[...] = jnp.maximum(p0_ref[...] + p1_ref[...] - h_ref[...], 0.0)

    return pl.pallas_call(
        body,
        grid=(_GRID,),
        in_specs=[pl.BlockSpec((_BLK, D), lambda i: (i, 0)),
                  pl.BlockSpec((_BLK, D), lambda i: (i, 0)),
                  pl.BlockSpec((_BLK, D), lambda i: (i, 0))],
        out_specs=pl.BlockSpec((_BLK, D), lambda i: (i, 0)),
        out_shape=jax.ShapeDtypeStruct((N_NODES, D), jnp.float32),
    )(p0, p1, h)


def kernel(edge_index, node_feats, W1, b1, W2, b2):
    src = edge_index[0].astype(jnp.int32)
    dst = edge_index[1].astype(jnp.int32)
    h1 = _mm(node_feats, W1, b1)
    p = _edge_agg(src, dst, h1)
    h2 = _combine_mm(p[0], p[1], h1, W2, b2)
    q = _edge_agg(src, dst, h2)
    return _combine_relu(q[0], q[1], h2)
